# baseline (speedup 1.0000x reference)
import functools
import jax
import jax.numpy as jnp
from jax import lax
from jax.experimental import pallas as pl
from jax.experimental.pallas import tpu as pltpu
from jax.experimental.pallas import tpu_sc as plsc

_NC = 2
_NS = 16
_L = 16
_OG = 8
_BG = 4
_CPB = 4


def _sc_body(f, g, opg, bc, nbv, xc_ref, cre_ref, gvec_ref, bvec_ref,
             out_ref, cv, xbuf, yv, gv, bv):
    wid = lax.axis_index("s") * _NC + lax.axis_index("c")
    og = lax.rem(wid, _OG)
    bg = lax.div(wid, _OG)

    pltpu.sync_copy(cre_ref.at[og], cv)
    pltpu.sync_copy(gvec_ref, gv)
    pltpu.sync_copy(bvec_ref, bv)

    ghead = gv[pl.ds(0, _L)]
    gtail = gv[pl.ds(g - _L, _L)]
    gmin = ghead[0]
    invh = (g - 1.0) / jnp.full((_L,), gtail[_L - 1] - gmin, jnp.float32)
    lanes = lax.iota(jnp.int32, _L)
    xlane = lanes * f
    ob = og * opg
    bias_init = tuple(
        plsc.load_gather(bv, [jnp.full((_L,), ob + j, jnp.int32)])
        for j in range(opg))

    def f_body(fi, accs):
        xv = plsc.load_gather(xbuf, [f_body_base[0] + fi])
        u = (xv - gmin) * invh
        it = u.astype(jnp.int32)
        ic = it + jnp.where(u > it.astype(jnp.float32), 1, 0)
        idx = jnp.clip(ic, 1, g - 1)
        i0 = idx - 1
        t = u - i0.astype(jnp.float32)
        ib0 = i0 * opg + fi * (g * opg)
        ib1 = ib0 + opg
        out = []
        for j in range(opg):
            c0 = plsc.load_gather(cv, [ib0 + j])
            c1 = plsc.load_gather(cv, [ib1 + j])
            out.append(accs[j] + (c0 + t * (c1 - c0)))
        return tuple(out)

    f_body_base = [None]

    def bvec_body(bi, carry):
        bb = bi * _L
        f_body_base[0] = bb * f + xlane
        accs = lax.fori_loop(0, f, f_body, bias_init)
        for j in range(opg):
            yv[j, pl.ds(bb, _L)] = accs[j]
        return carry

    def chunk_body(ci, carry):
        cg = bg * _CPB + ci
        pltpu.sync_copy(xc_ref.at[cg], xbuf)
        lax.fori_loop(0, bc // _L, bvec_body, 0)
        pltpu.sync_copy(yv, out_ref.at[cg, og])
        return carry

    lax.fori_loop(0, _CPB, chunk_body, 0)


def kernel(x, coeff, bias, grid):
    x = x.astype(jnp.float32)
    if x.ndim != 2:
        x = x.reshape(x.shape[0], -1)
    b, f = x.shape
    g = grid.shape[0]
    o = coeff.shape[-1]
    opg = o // _OG
    nch = _BG * _CPB
    bc = b // nch

    xc = x.reshape(nch, bc * f)
    cre = (coeff.astype(jnp.float32).reshape(f, g, _OG, opg)
           .transpose(2, 0, 1, 3).reshape(_OG, f * g * opg))
    gvec = grid.astype(jnp.float32)
    bvec = bias.astype(jnp.float32)

    mesh = plsc.VectorSubcoreMesh(core_axis_name="c", subcore_axis_name="s")
    run = functools.partial(
        pl.kernel,
        mesh=mesh,
        compiler_params=pltpu.CompilerParams(needs_layout_passes=False),
        out_type=jax.ShapeDtypeStruct((nch, _OG, opg, bc), jnp.float32),
        scratch_types=[
            pltpu.VMEM((f * g * opg,), jnp.float32),
            pltpu.VMEM((bc * f,), jnp.float32),
            pltpu.VMEM((opg, bc), jnp.float32),
            pltpu.VMEM((g,), jnp.float32),
            pltpu.VMEM((o,), jnp.float32),
        ],
    )(functools.partial(_sc_body, f, g, opg, bc, _OG * opg))
    yblk = run(xc, cre, gvec, bvec)
    return yblk.transpose(0, 3, 1, 2).reshape(b, o)
